# baseline (speedup 1.0000x reference)
import functools

import jax
import jax.numpy as jnp
from jax import lax
from jax.experimental import pallas as pl
from jax.experimental.pallas import tpu as pltpu
from jax.experimental.pallas import tpu_sc as plsc

_N = 6000
_E = 96000
_NC = 2
_NS = 16
_NW = _NC * _NS
_EPT = _E // _NW
_C = 120
_NCH = _EPT // _C
_NP = 6016
_RPT = _NP // _NS

_mesh = plsc.VectorSubcoreMesh(core_axis_name="c", subcore_axis_name="s")


def _wid():
    return lax.axis_index("s") * _NC + lax.axis_index("c")


def _sc_degrees(srcr, dstr, ones16, zeros16):
    @functools.partial(
        pl.kernel, mesh=_mesh,
        compiler_params=pltpu.CompilerParams(use_tc_tiling_on_sc=False),
        out_type=(jax.ShapeDtypeStruct((_NC, _NP, 16), jnp.float32),
                  jax.ShapeDtypeStruct((_NC, _NP, 16), jnp.float32)),
        scratch_types=[pltpu.VMEM((_NCH, _C), jnp.int32),
                       pltpu.VMEM((_NCH, _C), jnp.int32),
                       pltpu.VMEM((_C, 16), jnp.float32),
                       pltpu.VMEM_SHARED((_NP, 16), jnp.float32),
                       pltpu.VMEM_SHARED((_NP, 16), jnp.float32),
                       pltpu.SemaphoreType.DMA, pltpu.SemaphoreType.DMA],
    )
    def k(src_h, dst_h, ones_h, zeros_h, degs_o, degd_o, ivs, ivd, vones,
          acca, accb, sa, sb):
        c = lax.axis_index("c")
        s = lax.axis_index("s")
        wid = _wid()
        rbase = s * _RPT
        pltpu.sync_copy(src_h.at[wid], ivs)
        pltpu.sync_copy(dst_h.at[wid], ivd)
        pltpu.sync_copy(ones_h, vones)
        pltpu.sync_copy(zeros_h, acca.at[pl.ds(rbase, _RPT)])
        pltpu.sync_copy(zeros_h, accb.at[pl.ds(rbase, _RPT)])
        plsc.subcore_barrier()

        def fire(j, carry):
            pltpu.async_copy(vones, acca.at[ivs.at[j]], sa, add=True)
            pltpu.async_copy(vones, accb.at[ivd.at[j]], sb, add=True)
            return carry
        lax.fori_loop(0, _NCH, fire, 0)

        def drain(j, carry):
            pltpu.make_async_copy(vones, acca.at[pl.ds(0, _C)], sa).wait()
            pltpu.make_async_copy(vones, accb.at[pl.ds(0, _C)], sb).wait()
            return carry
        lax.fori_loop(0, _NCH, drain, 0)
        plsc.subcore_barrier()
        pltpu.sync_copy(acca.at[pl.ds(rbase, _RPT)],
                        degs_o.at[c, pl.ds(rbase, _RPT)])
        pltpu.sync_copy(accb.at[pl.ds(rbase, _RPT)],
                        degd_o.at[c, pl.ds(rbase, _RPT)])
    return k(srcr, dstr, ones16, zeros16)


def _sc_gather_scatter(table, gidxr, sidxr, zeros, D):
    @functools.partial(
        pl.kernel, mesh=_mesh,
        compiler_params=pltpu.CompilerParams(use_tc_tiling_on_sc=False),
        out_type=jax.ShapeDtypeStruct((_NC, _NP, D), jnp.float32),
        scratch_types=[pltpu.VMEM((_NCH, _C), jnp.int32),
                       pltpu.VMEM((_NCH, _C), jnp.int32),
                       pltpu.VMEM((_C, D), jnp.float32),
                       pltpu.VMEM((_C, D), jnp.float32),
                       pltpu.VMEM_SHARED((_NP, D), jnp.float32),
                       pltpu.SemaphoreType.DMA, pltpu.SemaphoreType.DMA,
                       pltpu.SemaphoreType.DMA, pltpu.SemaphoreType.DMA],
    )
    def k(tab_h, gi_h, si_h, zeros_h, out_o, ivg, ivd, ra, rb, acc,
          sga, sgb, ssa, ssb):
        c = lax.axis_index("c")
        s = lax.axis_index("s")
        wid = _wid()
        rbase = s * _RPT
        pltpu.sync_copy(gi_h.at[wid], ivg)
        pltpu.sync_copy(si_h.at[wid], ivd)
        pltpu.sync_copy(zeros_h, acc.at[pl.ds(rbase, _RPT)])
        plsc.subcore_barrier()
        r = (ra, rb)
        sg = (sga, sgb)
        ss = (ssa, ssb)

        def fire_gather(j, k_):
            pltpu.async_copy(tab_h.at[ivg.at[j]], r[k_], sg[k_])

        def drain_scatter(k_):
            pltpu.make_async_copy(r[k_], acc.at[pl.ds(0, _C)], ss[k_]).wait()

        def finish(j, k_):
            pltpu.make_async_copy(tab_h.at[ivg.at[j]], r[k_], sg[k_]).wait()
            pltpu.async_copy(r[k_], acc.at[ivd.at[j]], ss[k_], add=True)

        fire_gather(0, 0)
        fire_gather(1, 1)
        finish(0, 0)

        def body(j0, carry):
            j1 = 2 * j0 + 1
            drain_scatter(0)
            fire_gather(j1 + 1, 0)
            finish(j1, 1)
            drain_scatter(1)
            fire_gather(j1 + 2, 1)
            finish(j1 + 1, 0)
            return carry
        lax.fori_loop(0, (_NCH - 3) // 2, body, 0)
        drain_scatter(0)
        fire_gather(_NCH - 1, 0)
        finish(_NCH - 2, 1)
        finish(_NCH - 1, 0)
        drain_scatter(0)
        drain_scatter(1)
        plsc.subcore_barrier()
        pltpu.sync_copy(acc.at[pl.ds(rbase, _RPT)],
                        out_o.at[c, pl.ds(rbase, _RPT)])
    return k(table, gidxr, sidxr, zeros)


def _sc_gather2(t1, t2, i1r, i2r, D, C):
    @functools.partial(
        pl.kernel, mesh=_mesh,
        out_type=(jax.ShapeDtypeStruct((_E, D), jnp.float32),
                  jax.ShapeDtypeStruct((_E, D), jnp.float32)),
        scratch_types=[pltpu.VMEM((_EPT // C, C), jnp.int32),
                       pltpu.VMEM((_EPT // C, C), jnp.int32),
                       pltpu.VMEM((C, D), jnp.float32),
                       pltpu.VMEM((C, D), jnp.float32),
                       pltpu.VMEM((C, D), jnp.float32),
                       pltpu.VMEM((C, D), jnp.float32),
                       pltpu.SemaphoreType.DMA, pltpu.SemaphoreType.DMA,
                       pltpu.SemaphoreType.DMA, pltpu.SemaphoreType.DMA,
                       pltpu.SemaphoreType.DMA, pltpu.SemaphoreType.DMA,
                       pltpu.SemaphoreType.DMA, pltpu.SemaphoreType.DMA],
    )
    def k(t1_h, t2_h, i1_h, i2_h, o1_o, o2_o, iv1, iv2, r1a, r1b, r2a, r2b,
          sg1a, sg1b, sg2a, sg2b, sw1a, sw1b, sw2a, sw2b):
        wid = _wid()
        base = wid * _EPT
        pltpu.sync_copy(i1_h.at[wid], iv1)
        pltpu.sync_copy(i2_h.at[wid], iv2)
        r1 = (r1a, r1b)
        r2 = (r2a, r2b)
        sg1 = (sg1a, sg1b)
        sg2 = (sg2a, sg2b)
        sw1 = (sw1a, sw1b)
        sw2 = (sw2a, sw2b)

        def chunk(j, k, drain):
            if drain:
                pltpu.make_async_copy(r1[k], o1_o.at[pl.ds(0, C)], sw1[k]).wait()
                pltpu.make_async_copy(r2[k], o2_o.at[pl.ds(0, C)], sw2[k]).wait()
            g1 = pltpu.async_copy(t1_h.at[iv1.at[j]], r1[k], sg1[k])
            g2 = pltpu.async_copy(t2_h.at[iv2.at[j]], r2[k], sg2[k])
            off = base + j * C
            g1.wait()
            pltpu.async_copy(r1[k], o1_o.at[pl.ds(off, C)], sw1[k])
            g2.wait()
            pltpu.async_copy(r2[k], o2_o.at[pl.ds(off, C)], sw2[k])

        chunk(0, 0, False)
        chunk(1, 1, False)

        def body(j0, carry):
            j = 2 * j0
            chunk(j, 0, True)
            chunk(j + 1, 1, True)
            return carry
        lax.fori_loop(1, (_EPT // C - 1) // 2, body, 0)
        chunk(_EPT // C - 1, 0, True)
        pltpu.make_async_copy(r1[0], o1_o.at[pl.ds(0, C)], sw1[0]).wait()
        pltpu.make_async_copy(r2[0], o2_o.at[pl.ds(0, C)], sw2[0]).wait()
        pltpu.make_async_copy(r1[1], o1_o.at[pl.ds(0, C)], sw1[1]).wait()
        pltpu.make_async_copy(r2[1], o2_o.at[pl.ds(0, C)], sw2[1]).wait()
    return k(t1, t2, i1r, i2r)


def _sc_scatter_wex(wex, sidxr, zeros, W, SPLIT, NCH=_NCH):
    DS = W // SPLIT
    EPT = NCH * _C
    M = _NW * EPT

    @functools.partial(
        pl.kernel, mesh=_mesh,
        out_type=jax.ShapeDtypeStruct((_NC, _NP, W), jnp.float32),
        scratch_types=[pltpu.VMEM((NCH, _C), jnp.int32),
                       pltpu.VMEM((_C, DS), jnp.float32),
                       pltpu.VMEM((_C, DS), jnp.float32),
                       pltpu.VMEM_SHARED((_NP, DS), jnp.float32),
                       pltpu.SemaphoreType.DMA, pltpu.SemaphoreType.DMA,
                       pltpu.SemaphoreType.DMA, pltpu.SemaphoreType.DMA],
    )
    def k(w_h, si_h, zeros_h, out_o, iv, vwa, vwb, accd, sva, svb, ssa, ssb):
        c = lax.axis_index("c")
        s_ = lax.axis_index("s")
        wid = s_ * _NC + c
        base = wid * EPT
        rbase = s_ * _RPT
        pltpu.sync_copy(si_h.at[wid], iv)
        vw = (vwa, vwb)
        sv = (sva, svb)
        ss = (ssa, ssb)
        for sp in range(SPLIT):
            pltpu.sync_copy(zeros_h, accd.at[pl.ds(rbase, _RPT)])
            plsc.subcore_barrier()

            def fire_load(j, k):
                off = base + j * _C
                pltpu.async_copy(
                    w_h.at[pl.ds(off, _C), pl.ds(sp * DS, DS)], vw[k], sv[k])

            def drain_scatter(k):
                pltpu.make_async_copy(
                    vw[k], accd.at[pl.ds(0, _C)], ss[k]).wait()

            def finish(j, k):
                pltpu.make_async_copy(
                    w_h.at[pl.ds(0, _C), pl.ds(sp * DS, DS)], vw[k],
                    sv[k]).wait()
                pltpu.async_copy(vw[k], accd.at[iv.at[j]], ss[k], add=True)

            fire_load(0, 0)
            fire_load(1, 1)
            finish(0, 0)

            def body(j0, carry):
                j1 = 2 * j0 + 1
                drain_scatter(0)
                fire_load(j1 + 1, 0)
                finish(j1, 1)
                drain_scatter(1)
                fire_load(j1 + 2, 1)
                finish(j1 + 1, 0)
                return carry
            lax.fori_loop(0, (NCH - 3) // 2, body, 0)
            if NCH % 2 == 1:
                drain_scatter(0)
                fire_load(NCH - 1, 0)
                finish(NCH - 2, 1)
                finish(NCH - 1, 0)
            else:
                drain_scatter(0)
                fire_load(NCH - 2, 0)
                finish(NCH - 3, 1)
                drain_scatter(1)
                fire_load(NCH - 1, 1)
                finish(NCH - 2, 0)
                finish(NCH - 1, 1)
            drain_scatter(0)
            drain_scatter(1)
            plsc.subcore_barrier()
            pltpu.sync_copy(accd.at[pl.ds(rbase, _RPT)],
                            out_o.at[c, pl.ds(rbase, _RPT), pl.ds(sp * DS, DS)])
            if sp < SPLIT - 1:
                plsc.subcore_barrier()
    return k(wex, sidxr, zeros)


def _dot(a, b):
    return jnp.dot(a, b, preferred_element_type=jnp.float32)


def _pack_bf16(v):
    h = v.shape[1] // 2
    vi = lax.bitcast_convert_type(v, jnp.int32)
    vr = vi + (0x7FFF + (lax.shift_right_logical(vi, 16) & 1))
    hi = vr[:, 0:h] & jnp.int32(-65536)
    lo = lax.shift_right_logical(vr[:, h:2 * h], 16)
    return lax.bitcast_convert_type(hi | lo, jnp.float32)


def _unpack_bf16(pv):
    pi = lax.bitcast_convert_type(pv, jnp.int32)
    a = lax.bitcast_convert_type(pi & jnp.int32(-65536), jnp.float32)
    b = lax.bitcast_convert_type(lax.shift_left(pi, 16), jnp.float32)
    return jnp.concatenate([a, b], axis=1)


def _tc_fusion(text, audio, vision, p):
    B = 600

    def body(t_r, a_r, v_r, tw, tb, aw, ab, vw, vb, wf, bf1, bf2, wb, bb1, bb2,
             stack_o, xf_o, xb_o):
        t = _dot(t_r[...], tw[...]) + tb[...]
        a = _dot(a_r[...], aw[...]) + ab[...]
        v = _dot(v_r[...], vw[...]) + vb[...]
        stack = jnp.concatenate([t, a, v], axis=1)
        stack_o[...] = stack
        xf_o[...] = _dot(stack, wf[...]) + bf1[...] + bf2[...]
        xb_o[...] = _dot(stack, wb[...]) + bb1[...] + bb2[...]

    full = lambda shape: pl.BlockSpec(shape, lambda i: (0,) * len(shape))
    row = lambda d: pl.BlockSpec((B, d), lambda i: (i, 0))
    vec = lambda d: pl.BlockSpec((d,), lambda i: (0,))
    return pl.pallas_call(
        body,
        grid=(_N // B,),
        in_specs=[row(1024), row(512), row(1024),
                  full((1024, 64)), vec(64), full((512, 64)), vec(64),
                  full((1024, 64)), vec(64),
                  full((192, 32)), vec(32), vec(32),
                  full((192, 32)), vec(32), vec(32)],
        out_specs=[row(192), row(32), row(32)],
        out_shape=[jax.ShapeDtypeStruct((_N, 192), jnp.float32),
                   jax.ShapeDtypeStruct((_N, 32), jnp.float32),
                   jax.ShapeDtypeStruct((_N, 32), jnp.float32)],
    )(text, audio, vision,
      p['textW'], p['textB'], p['audioW'], p['audioB'], p['visionW'], p['visionB'],
      p['lfWih'], p['lfbih'], p['lfbhh'], p['lbWih'], p['lbbih'], p['lbbhh'])


def _tc_lstm(xf, xb, whf, whb):
    T, Bb, H = 120, 50, 8

    def body(xf_r, xb_r, wf_r, wb_r, out_o):
        def gates(g, c):
            i, f, gg, o = jnp.split(g, 4, axis=-1)
            c2 = jax.nn.sigmoid(f) * c + jax.nn.sigmoid(i) * jnp.tanh(gg)
            h2 = jax.nn.sigmoid(o) * jnp.tanh(c2)
            return h2, c2

        def fstep(t, hc):
            h, c = hc
            h2, c2 = gates(xf_r[t] + _dot(h, wf_r[...]), c)
            out_o[t, :, 0:8] = h2
            return (h2, c2)

        def bstep(t, hc):
            h, c = hc
            tt = T - 1 - t
            h2, c2 = gates(xb_r[tt] + _dot(h, wb_r[...]), c)
            out_o[tt, :, 8:16] = h2
            return (h2, c2)

        z = (jnp.zeros((Bb, H), jnp.float32), jnp.zeros((Bb, H), jnp.float32))
        lax.fori_loop(0, T, fstep, z)
        lax.fori_loop(0, T, bstep, z)

    return pl.pallas_call(
        body,
        out_shape=jax.ShapeDtypeStruct((T, Bb, 16), jnp.float32),
    )(xf, xb, whf, whb)


def _deg_norm(dp):
    deg = dp[0, :, 0:1] + dp[1, :, 0:1]
    return jnp.where(deg > 0, lax.rsqrt(jnp.maximum(deg, 1.0)), 0.0)


def _tc_scale_h(h, degs):
    B = 600

    def body(h_r, d_r, out_o):
        out_o[...] = h_r[...] * _deg_norm(d_r)

    return pl.pallas_call(
        body,
        grid=(_N // B,),
        in_specs=[pl.BlockSpec((B, 192), lambda i: (i, 0)),
                  pl.BlockSpec((2, B, 16), lambda i: (0, i, 0))],
        out_specs=pl.BlockSpec((B, 192), lambda i: (i, 0)),
        out_shape=jax.ShapeDtypeStruct((_N, 192), jnp.float32),
    )(h, degs)


def _tc_gcn_dense(h, aggp, degd, p):
    B = 600

    def body(h_r, a_r, d_r, impW, impB, decW, decB, mask,
             g2Wl, g2Wr, g0Wl, g0Wr, g0res,
             fs2_o, fd2_o, fs0_o, fd0_o, res0_o):
        agg = (a_r[0] + a_r[1]) * _deg_norm(d_r)
        h1 = _dot(agg, impW[...]) + impB[...]
        h1 = _dot(h1, decW[...]) + decB[...]
        hm = 0.1 * h_r[...] + 0.9 * h1
        l1 = jnp.sum(jnp.abs(hm), axis=1, keepdims=True)
        hm = hm / jnp.maximum(l1, 1e-12) * mask[...]
        zpad = jnp.zeros((hm.shape[0], 64), jnp.float32)
        fs2_o[...] = jnp.concatenate([_dot(hm, g2Wl[...]), zpad], axis=1)
        fd2_o[...] = jnp.concatenate([_dot(hm, g2Wr[...]), zpad], axis=1)
        fs0_o[...] = _pack_bf16(_dot(hm, g0Wl[...]))
        fd0_o[...] = _pack_bf16(_dot(hm, g0Wr[...]))
        res0_o[...] = _dot(hm, g0res[...])

    full = lambda shape: pl.BlockSpec(shape, lambda i: (0,) * len(shape))
    row = lambda d: pl.BlockSpec((B, d), lambda i: (i, 0))
    return pl.pallas_call(
        body,
        grid=(_N // B,),
        in_specs=[row(192),
                  pl.BlockSpec((2, B, 192), lambda i: (0, i, 0)),
                  pl.BlockSpec((2, B, 16), lambda i: (0, i, 0)),
                  full((192, 192)), pl.BlockSpec((192,), lambda i: (0,)),
                  full((192, 192)), pl.BlockSpec((192,), lambda i: (0,)),
                  pl.BlockSpec((192,), lambda i: (0,)),
                  full((192, 64)), full((192, 64)),
                  full((192, 512)), full((192, 512)), full((192, 512))],
        out_specs=[row(128), row(128), row(256), row(256), row(512)],
        out_shape=[jax.ShapeDtypeStruct((_N, 128), jnp.float32),
                   jax.ShapeDtypeStruct((_N, 128), jnp.float32),
                   jax.ShapeDtypeStruct((_N, 256), jnp.float32),
                   jax.ShapeDtypeStruct((_N, 256), jnp.float32),
                   jax.ShapeDtypeStruct((_N, 512), jnp.float32)],
    )(h, aggp, degd, p['impW'], p['impB'], p['decW'], p['decB'], p['mask'],
      p['g2Wl'], p['g2Wr'], p['g0Wl'], p['g0Wr'], p['g0res'])


def _tc_edge(fsg, fdg, A, R, D, DW, W, BE, packed=False, ne=_E, goff=0):
    def body(fs_r, fd_r, a_r, r_r, wex_o):
        if packed:
            fs = _unpack_bf16(fs_r[...])
            e = fs + _unpack_bf16(fd_r[...])
        else:
            fs = fs_r[:, 0:D]
            e = fs + fd_r[:, 0:D]
        e = jnp.where(e > 0, e, 0.2 * e)
        ex = jnp.exp(_dot(e, a_r[...]))
        wex_o[:, 0:D] = fs * _dot(ex, r_r[...])
        wex_o[:, D:D + 16] = ex
        wex_o[:, D + 16:W] = jnp.zeros((BE, W - D - 16), jnp.float32)

    full = lambda shape: pl.BlockSpec(shape, lambda i: (0,) * len(shape))
    rowi = lambda d: pl.BlockSpec((BE, d), lambda i: (i + goff, 0))
    row = lambda d: pl.BlockSpec((BE, d), lambda i: (i, 0))
    return pl.pallas_call(
        body,
        grid=(ne // BE,),
        in_specs=[rowi(DW), rowi(DW), full((D, 16)), full((16, D))],
        out_specs=row(W),
        out_shape=jax.ShapeDtypeStruct((ne, W), jnp.float32),
    )(fsg, fdg, A, R)


def _tc_finish(outp, R, D, W, res=None, bias=None, relu=False,
               proj=None, proj_pad=None):
    B = 600
    proj = proj or []
    proj_pad = proj_pad or [w.shape[1] for w in proj]
    nproj = len(proj)

    full = lambda shape: pl.BlockSpec(shape, lambda i: (0,) * len(shape))
    row = lambda d: pl.BlockSpec((B, d), lambda i: (i, 0))
    outps = outp if isinstance(outp, (list, tuple)) else [outp]
    in_specs = [pl.BlockSpec((2, B, W), lambda i: (0, i, 0))
                for _ in outps] + [full((16, D))]
    args = list(outps) + [R]
    if res is not None:
        in_specs.append(row(D)); args.append(res)
    if bias is not None:
        in_specs.append(pl.BlockSpec((D,), lambda i: (0,))); args.append(bias)
    for w in proj:
        in_specs.append(full(w.shape)); args.append(w)
    out_specs = [row(D)] + [row(pw) for pw in proj_pad]
    out_shape = [jax.ShapeDtypeStruct((_N, D), jnp.float32)] +                 [jax.ShapeDtypeStruct((_N, pw), jnp.float32) for pw in proj_pad]

    def body(*refs):
        nin = len(in_specs)
        ins, outs = refs[:nin], refs[nin:]
        np_ = len(outps)
        r_r = ins[np_]
        idx = np_ + 1
        dn = sum(ins[q][pp, :, D:D + 16] for q in range(np_) for pp in (0, 1))
        num = sum(ins[q][pp, :, 0:D] for q in range(np_) for pp in (0, 1))
        out = num / (_dot(dn, r_r[...]) + 1e-16)
        if res is not None:
            out = out + ins[idx][...]
            idx += 1
        if bias is not None:
            out = out + ins[idx][...]
            idx += 1
        if relu:
            out = jnp.maximum(out, 0.0)
        outs[0][...] = out
        for j in range(nproj):
            pr = _dot(out, ins[idx + j][...])
            if proj_pad[j] > pr.shape[1]:
                pr = jnp.concatenate(
                    [pr, jnp.zeros((pr.shape[0], proj_pad[j] - pr.shape[1]),
                                   jnp.float32)], axis=1)
            outs[1 + j][...] = pr

    return pl.pallas_call(
        body,
        grid=(_N // B,),
        in_specs=in_specs,
        out_specs=out_specs if len(out_specs) > 1 else out_specs[0],
        out_shape=out_shape if len(out_shape) > 1 else out_shape[0],
    )(*args)


def _tc_final(h, nf, h3, W, b):
    B = 600

    def body(h_r, n_r, h3_r, w_r, b_r, out_o):
        cat = jnp.concatenate([h_r[...], n_r[...], h3_r[...]], axis=1)
        out_o[...] = _dot(cat, w_r[...]) + b_r[...]

    full = lambda shape: pl.BlockSpec(shape, lambda i: (0,) * len(shape))
    row = lambda d: pl.BlockSpec((B, d), lambda i: (i, 0))
    return pl.pallas_call(
        body,
        grid=(_N // B,),
        in_specs=[row(64), row(16), row(64), full((144, 6)),
                  pl.BlockSpec((6,), lambda i: (0,))],
        out_specs=row(6),
        out_shape=jax.ShapeDtypeStruct((_N, 6), jnp.float32),
    )(h, nf, h3, W, b)


def _attn_mats(attn, heads, d):
    D = heads * d
    eye = jnp.eye(heads, dtype=jnp.float32)
    A = (attn[:, :, None] * eye[:, None, :]).reshape(D, heads)
    R = jnp.repeat(eye, d, axis=1)
    return A, R


def kernel(text, audio, vision, oText, oAudio, oVision, edge_index, params):
    p = params
    src = edge_index[0]
    dst = edge_index[1]
    ones16 = jnp.ones((_C, 16), jnp.float32)
    src2 = src.reshape(_NW, _NCH, _C)
    dst2 = dst.reshape(_NW, _NCH, _C)
    z16 = jnp.zeros((_RPT, 16), jnp.float32)
    z192 = jnp.zeros((_RPT, 192), jnp.float32)
    z128 = jnp.zeros((_RPT, 128), jnp.float32)

    degs, degd = _sc_degrees(src2, dst2, ones16, z16)

    stack, xf, xb = _tc_fusion(text, audio, vision, p)
    xf_t = xf.reshape(50, 120, 32).transpose(1, 0, 2)
    xb_t = xb.reshape(50, 120, 32).transpose(1, 0, 2)
    hcat = _tc_lstm(xf_t, xb_t, p['lfWhh'], p['lbWhh'])
    newF = hcat.transpose(1, 0, 2).reshape(_N, 16)

    hs = _tc_scale_h(stack, degs)
    aggp = _sc_gather_scatter(hs, src2, dst2, z192, 192)
    fs2, fd2, fs0, fd0, res0 = _tc_gcn_dense(stack, aggp, degd, p)

    A2, R2 = _attn_mats(p['g2attn'], 16, 4)
    fsg2, fdg2 = _sc_gather2(fs2, fd2, src2, dst2, 128, _C)

    A0, R0 = _attn_mats(p['g0attn'], 16, 32)
    A0 = A0 + newF[0, 0] * 0.0
    fsg, fdg = _sc_gather2(fs0, fd0, src2, dst2, 256, _C)
    NA = 12 * _C * _NW
    wexA = _tc_edge(fsg, fdg, A0, R0, 512, 256, 640, 960, packed=True,
                    ne=NA, goff=0)
    wexB = _tc_edge(fsg, fdg, A0, R0, 512, 256, 640, 960, packed=True,
                    ne=_E - NA, goff=NA // 960)
    dstA = dst[0:NA].reshape(_NW, 12, _C)
    dstB = dst[NA:].reshape(_NW, 13, _C)
    outpA = _sc_scatter_wex(wexA, dstA, z128, 640, 5, NCH=12)
    outpB = _sc_scatter_wex(wexB, dstB, z128, 640, 5, NCH=13)
    h0, fs1, fd1, res1 = _tc_finish([outpA, outpB], R0, 512, 640,
                                    res=res0, bias=p['g0bias'], relu=True,
                                    proj=[p['g1Wl'], p['g1Wr'], p['g1res']],
                                    proj_pad=[128, 128, 64])

    wex2 = _tc_edge(fsg2, fdg2, A2, R2, 64, 128, 128, 3000)
    outp2 = _sc_scatter_wex(wex2, dst2, z128, 128, 1)
    h3 = _tc_finish(outp2, R2, 64, 128)

    A1, R1 = _attn_mats(p['g1attn'], 16, 4)
    fsg, fdg = _sc_gather2(fs1, fd1, src2, dst2, 128, _C)
    wex = _tc_edge(fsg, fdg, A1, R1, 64, 128, 128, 3000)
    outp = _sc_scatter_wex(wex, dst2, z128, 128, 1)
    hfin = _tc_finish(outp, R1, 64, 128, res=res1, bias=p['g1bias'], relu=True)

    return _tc_final(hfin, newF, h3, p['linW'], p['linB'])


# speedup vs baseline: 1.0570x; 1.0570x over previous
import functools

import jax
import jax.numpy as jnp
from jax import lax
from jax.experimental import pallas as pl
from jax.experimental.pallas import tpu as pltpu
from jax.experimental.pallas import tpu_sc as plsc

_N = 6000
_E = 96000
_NC = 2
_NS = 16
_NW = _NC * _NS
_EPT = _E // _NW
_C = 120
_NCH = _EPT // _C
_NP = 6016
_RPT = _NP // _NS

_mesh = plsc.VectorSubcoreMesh(core_axis_name="c", subcore_axis_name="s")


def _wid():
    return lax.axis_index("s") * _NC + lax.axis_index("c")


def _sc_degrees(srcr, dstr, ones16, zeros16):
    @functools.partial(
        pl.kernel, mesh=_mesh,
        compiler_params=pltpu.CompilerParams(use_tc_tiling_on_sc=False),
        out_type=(jax.ShapeDtypeStruct((_NC, _NP, 16), jnp.float32),
                  jax.ShapeDtypeStruct((_NC, _NP, 16), jnp.float32)),
        scratch_types=[pltpu.VMEM((_NCH, _C), jnp.int32),
                       pltpu.VMEM((_NCH, _C), jnp.int32),
                       pltpu.VMEM((_C, 16), jnp.float32),
                       pltpu.VMEM_SHARED((_NP, 16), jnp.float32),
                       pltpu.VMEM_SHARED((_NP, 16), jnp.float32),
                       pltpu.SemaphoreType.DMA, pltpu.SemaphoreType.DMA],
    )
    def k(src_h, dst_h, ones_h, zeros_h, degs_o, degd_o, ivs, ivd, vones,
          acca, accb, sa, sb):
        c = lax.axis_index("c")
        s = lax.axis_index("s")
        wid = _wid()
        rbase = s * _RPT
        pltpu.sync_copy(src_h.at[wid], ivs)
        pltpu.sync_copy(dst_h.at[wid], ivd)
        pltpu.sync_copy(ones_h, vones)
        pltpu.sync_copy(zeros_h, acca.at[pl.ds(rbase, _RPT)])
        pltpu.sync_copy(zeros_h, accb.at[pl.ds(rbase, _RPT)])
        plsc.subcore_barrier()

        def fire(j, carry):
            pltpu.async_copy(vones, acca.at[ivs.at[j]], sa, add=True)
            pltpu.async_copy(vones, accb.at[ivd.at[j]], sb, add=True)
            return carry
        lax.fori_loop(0, _NCH, fire, 0)

        def drain(j, carry):
            pltpu.make_async_copy(vones, acca.at[pl.ds(0, _C)], sa).wait()
            pltpu.make_async_copy(vones, accb.at[pl.ds(0, _C)], sb).wait()
            return carry
        lax.fori_loop(0, _NCH, drain, 0)
        plsc.subcore_barrier()
        pltpu.sync_copy(acca.at[pl.ds(rbase, _RPT)],
                        degs_o.at[c, pl.ds(rbase, _RPT)])
        pltpu.sync_copy(accb.at[pl.ds(rbase, _RPT)],
                        degd_o.at[c, pl.ds(rbase, _RPT)])
    return k(srcr, dstr, ones16, zeros16)


def _sc_gather_scatter(table, gidxr, sidxr, zeros, D):
    @functools.partial(
        pl.kernel, mesh=_mesh,
        compiler_params=pltpu.CompilerParams(use_tc_tiling_on_sc=False),
        out_type=jax.ShapeDtypeStruct((_NC, _NP, D), jnp.float32),
        scratch_types=[pltpu.VMEM((_NCH, _C), jnp.int32),
                       pltpu.VMEM((_NCH, _C), jnp.int32),
                       pltpu.VMEM((_C, D), jnp.float32),
                       pltpu.VMEM((_C, D), jnp.float32),
                       pltpu.VMEM_SHARED((_NP, D), jnp.float32),
                       pltpu.SemaphoreType.DMA, pltpu.SemaphoreType.DMA,
                       pltpu.SemaphoreType.DMA, pltpu.SemaphoreType.DMA],
    )
    def k(tab_h, gi_h, si_h, zeros_h, out_o, ivg, ivd, ra, rb, acc,
          sga, sgb, ssa, ssb):
        c = lax.axis_index("c")
        s = lax.axis_index("s")
        wid = _wid()
        rbase = s * _RPT
        pltpu.sync_copy(gi_h.at[wid], ivg)
        pltpu.sync_copy(si_h.at[wid], ivd)
        pltpu.sync_copy(zeros_h, acc.at[pl.ds(rbase, _RPT)])
        plsc.subcore_barrier()
        r = (ra, rb)
        sg = (sga, sgb)
        ss = (ssa, ssb)

        def fire_gather(j, k_):
            pltpu.async_copy(tab_h.at[ivg.at[j]], r[k_], sg[k_])

        def drain_scatter(k_):
            pltpu.make_async_copy(r[k_], acc.at[pl.ds(0, _C)], ss[k_]).wait()

        def finish(j, k_):
            pltpu.make_async_copy(tab_h.at[ivg.at[j]], r[k_], sg[k_]).wait()
            pltpu.async_copy(r[k_], acc.at[ivd.at[j]], ss[k_], add=True)

        fire_gather(0, 0)
        fire_gather(1, 1)
        finish(0, 0)

        def body(j0, carry):
            j1 = 2 * j0 + 1
            drain_scatter(0)
            fire_gather(j1 + 1, 0)
            finish(j1, 1)
            drain_scatter(1)
            fire_gather(j1 + 2, 1)
            finish(j1 + 1, 0)
            return carry
        lax.fori_loop(0, (_NCH - 3) // 2, body, 0)
        drain_scatter(0)
        fire_gather(_NCH - 1, 0)
        finish(_NCH - 2, 1)
        finish(_NCH - 1, 0)
        drain_scatter(0)
        drain_scatter(1)
        plsc.subcore_barrier()
        pltpu.sync_copy(acc.at[pl.ds(rbase, _RPT)],
                        out_o.at[c, pl.ds(rbase, _RPT)])
    return k(table, gidxr, sidxr, zeros)


def _sc_gather2(t1, t2, i1r, i2r, D, C):
    @functools.partial(
        pl.kernel, mesh=_mesh,
        out_type=(jax.ShapeDtypeStruct((_E, D), jnp.float32),
                  jax.ShapeDtypeStruct((_E, D), jnp.float32)),
        scratch_types=[pltpu.VMEM((_EPT // C, C), jnp.int32),
                       pltpu.VMEM((_EPT // C, C), jnp.int32),
                       pltpu.VMEM((C, D), jnp.float32),
                       pltpu.VMEM((C, D), jnp.float32),
                       pltpu.VMEM((C, D), jnp.float32),
                       pltpu.VMEM((C, D), jnp.float32),
                       pltpu.SemaphoreType.DMA, pltpu.SemaphoreType.DMA,
                       pltpu.SemaphoreType.DMA, pltpu.SemaphoreType.DMA,
                       pltpu.SemaphoreType.DMA, pltpu.SemaphoreType.DMA,
                       pltpu.SemaphoreType.DMA, pltpu.SemaphoreType.DMA],
    )
    def k(t1_h, t2_h, i1_h, i2_h, o1_o, o2_o, iv1, iv2, r1a, r1b, r2a, r2b,
          sg1a, sg1b, sg2a, sg2b, sw1a, sw1b, sw2a, sw2b):
        wid = _wid()
        base = wid * _EPT
        pltpu.sync_copy(i1_h.at[wid], iv1)
        pltpu.sync_copy(i2_h.at[wid], iv2)
        r1 = (r1a, r1b)
        r2 = (r2a, r2b)
        sg1 = (sg1a, sg1b)
        sg2 = (sg2a, sg2b)
        sw1 = (sw1a, sw1b)
        sw2 = (sw2a, sw2b)

        def chunk(j, k, drain):
            if drain:
                pltpu.make_async_copy(r1[k], o1_o.at[pl.ds(0, C)], sw1[k]).wait()
                pltpu.make_async_copy(r2[k], o2_o.at[pl.ds(0, C)], sw2[k]).wait()
            g1 = pltpu.async_copy(t1_h.at[iv1.at[j]], r1[k], sg1[k])
            g2 = pltpu.async_copy(t2_h.at[iv2.at[j]], r2[k], sg2[k])
            off = base + j * C
            g1.wait()
            pltpu.async_copy(r1[k], o1_o.at[pl.ds(off, C)], sw1[k])
            g2.wait()
            pltpu.async_copy(r2[k], o2_o.at[pl.ds(off, C)], sw2[k])

        chunk(0, 0, False)
        chunk(1, 1, False)

        def body(j0, carry):
            j = 2 * j0
            chunk(j, 0, True)
            chunk(j + 1, 1, True)
            return carry
        lax.fori_loop(1, (_EPT // C - 1) // 2, body, 0)
        chunk(_EPT // C - 1, 0, True)
        pltpu.make_async_copy(r1[0], o1_o.at[pl.ds(0, C)], sw1[0]).wait()
        pltpu.make_async_copy(r2[0], o2_o.at[pl.ds(0, C)], sw2[0]).wait()
        pltpu.make_async_copy(r1[1], o1_o.at[pl.ds(0, C)], sw1[1]).wait()
        pltpu.make_async_copy(r2[1], o2_o.at[pl.ds(0, C)], sw2[1]).wait()
    return k(t1, t2, i1r, i2r)


def _sc_scatter_wex(wex, sidxr, zeros, W, SPLIT, NCH=_NCH):
    DS = W // SPLIT
    EPT = NCH * _C
    M = _NW * EPT

    @functools.partial(
        pl.kernel, mesh=_mesh,
        out_type=jax.ShapeDtypeStruct((_NC, _NP, W), jnp.float32),
        scratch_types=[pltpu.VMEM((NCH, _C), jnp.int32),
                       pltpu.VMEM((_C, DS), jnp.float32),
                       pltpu.VMEM((_C, DS), jnp.float32),
                       pltpu.VMEM_SHARED((_NP, DS), jnp.float32),
                       pltpu.SemaphoreType.DMA, pltpu.SemaphoreType.DMA,
                       pltpu.SemaphoreType.DMA, pltpu.SemaphoreType.DMA],
    )
    def k(w_h, si_h, zeros_h, out_o, iv, vwa, vwb, accd, sva, svb, ssa, ssb):
        c = lax.axis_index("c")
        s_ = lax.axis_index("s")
        wid = s_ * _NC + c
        base = wid * EPT
        rbase = s_ * _RPT
        pltpu.sync_copy(si_h.at[wid], iv)
        vw = (vwa, vwb)
        sv = (sva, svb)
        ss = (ssa, ssb)
        for sp in range(SPLIT):
            pltpu.sync_copy(zeros_h, accd.at[pl.ds(rbase, _RPT)])
            plsc.subcore_barrier()

            def fire_load(j, k):
                off = base + j * _C
                pltpu.async_copy(
                    w_h.at[pl.ds(off, _C), pl.ds(sp * DS, DS)], vw[k], sv[k])

            def drain_scatter(k):
                pltpu.make_async_copy(
                    vw[k], accd.at[pl.ds(0, _C)], ss[k]).wait()

            def finish(j, k):
                pltpu.make_async_copy(
                    w_h.at[pl.ds(0, _C), pl.ds(sp * DS, DS)], vw[k],
                    sv[k]).wait()
                pltpu.async_copy(vw[k], accd.at[iv.at[j]], ss[k], add=True)

            fire_load(0, 0)
            fire_load(1, 1)
            finish(0, 0)

            def body(j0, carry):
                j1 = 2 * j0 + 1
                drain_scatter(0)
                fire_load(j1 + 1, 0)
                finish(j1, 1)
                drain_scatter(1)
                fire_load(j1 + 2, 1)
                finish(j1 + 1, 0)
                return carry
            lax.fori_loop(0, (NCH - 3) // 2, body, 0)
            if NCH % 2 == 1:
                drain_scatter(0)
                fire_load(NCH - 1, 0)
                finish(NCH - 2, 1)
                finish(NCH - 1, 0)
            else:
                drain_scatter(0)
                fire_load(NCH - 2, 0)
                finish(NCH - 3, 1)
                drain_scatter(1)
                fire_load(NCH - 1, 1)
                finish(NCH - 2, 0)
                finish(NCH - 1, 1)
            drain_scatter(0)
            drain_scatter(1)
            plsc.subcore_barrier()
            pltpu.sync_copy(accd.at[pl.ds(rbase, _RPT)],
                            out_o.at[c, pl.ds(rbase, _RPT), pl.ds(sp * DS, DS)])
            if sp < SPLIT - 1:
                plsc.subcore_barrier()
    return k(wex, sidxr, zeros)


def _dot(a, b):
    return jnp.dot(a, b, preferred_element_type=jnp.float32)


def _pack_bf16(v):
    h = v.shape[1] // 2
    vi = lax.bitcast_convert_type(v, jnp.int32)
    vr = vi + (0x7FFF + (lax.shift_right_logical(vi, 16) & 1))
    hi = vr[:, 0:h] & jnp.int32(-65536)
    lo = lax.shift_right_logical(vr[:, h:2 * h], 16)
    return lax.bitcast_convert_type(hi | lo, jnp.float32)


def _unpack_bf16(pv):
    pi = lax.bitcast_convert_type(pv, jnp.int32)
    a = lax.bitcast_convert_type(pi & jnp.int32(-65536), jnp.float32)
    b = lax.bitcast_convert_type(lax.shift_left(pi, 16), jnp.float32)
    return jnp.concatenate([a, b], axis=1)


def _tc_fusion(text, audio, vision, p):
    B = 600

    def body(t_r, a_r, v_r, tw, tb, aw, ab, vw, vb, wf, bf1, bf2, wb, bb1, bb2,
             stack_o, xf_o, xb_o):
        t = _dot(t_r[...], tw[...]) + tb[...]
        a = _dot(a_r[...], aw[...]) + ab[...]
        v = _dot(v_r[...], vw[...]) + vb[...]
        stack = jnp.concatenate([t, a, v], axis=1)
        stack_o[...] = stack
        xf_o[...] = _dot(stack, wf[...]) + bf1[...] + bf2[...]
        xb_o[...] = _dot(stack, wb[...]) + bb1[...] + bb2[...]

    full = lambda shape: pl.BlockSpec(shape, lambda i: (0,) * len(shape))
    row = lambda d: pl.BlockSpec((B, d), lambda i: (i, 0))
    vec = lambda d: pl.BlockSpec((d,), lambda i: (0,))
    return pl.pallas_call(
        body,
        grid=(_N // B,),
        in_specs=[row(1024), row(512), row(1024),
                  full((1024, 64)), vec(64), full((512, 64)), vec(64),
                  full((1024, 64)), vec(64),
                  full((192, 32)), vec(32), vec(32),
                  full((192, 32)), vec(32), vec(32)],
        out_specs=[row(192), row(32), row(32)],
        out_shape=[jax.ShapeDtypeStruct((_N, 192), jnp.float32),
                   jax.ShapeDtypeStruct((_N, 32), jnp.float32),
                   jax.ShapeDtypeStruct((_N, 32), jnp.float32)],
    )(text, audio, vision,
      p['textW'], p['textB'], p['audioW'], p['audioB'], p['visionW'], p['visionB'],
      p['lfWih'], p['lfbih'], p['lfbhh'], p['lbWih'], p['lbbih'], p['lbbhh'])


def _tc_lstm(xf, xb, whf, whb):
    T, Bb, H = 120, 50, 8

    def body(xf_r, xb_r, wf_r, wb_r, out_o):
        def gates(g, c):
            i, f, gg, o = jnp.split(g, 4, axis=-1)
            c2 = jax.nn.sigmoid(f) * c + jax.nn.sigmoid(i) * jnp.tanh(gg)
            h2 = jax.nn.sigmoid(o) * jnp.tanh(c2)
            return h2, c2

        def fstep(t, hc):
            h, c = hc
            h2, c2 = gates(xf_r[t] + _dot(h, wf_r[...]), c)
            out_o[t, :, 0:8] = h2
            return (h2, c2)

        def bstep(t, hc):
            h, c = hc
            tt = T - 1 - t
            h2, c2 = gates(xb_r[tt] + _dot(h, wb_r[...]), c)
            out_o[tt, :, 8:16] = h2
            return (h2, c2)

        z = (jnp.zeros((Bb, H), jnp.float32), jnp.zeros((Bb, H), jnp.float32))
        lax.fori_loop(0, T, fstep, z)
        lax.fori_loop(0, T, bstep, z)

    return pl.pallas_call(
        body,
        out_shape=jax.ShapeDtypeStruct((T, Bb, 16), jnp.float32),
    )(xf, xb, whf, whb)


def _deg_norm(dp):
    deg = dp[0, :, 0:1] + dp[1, :, 0:1]
    return jnp.where(deg > 0, lax.rsqrt(jnp.maximum(deg, 1.0)), 0.0)


def _tc_scale_h(h, degs):
    B = 600

    def body(h_r, d_r, out_o):
        out_o[...] = h_r[...] * _deg_norm(d_r)

    return pl.pallas_call(
        body,
        grid=(_N // B,),
        in_specs=[pl.BlockSpec((B, 192), lambda i: (i, 0)),
                  pl.BlockSpec((2, B, 16), lambda i: (0, i, 0))],
        out_specs=pl.BlockSpec((B, 192), lambda i: (i, 0)),
        out_shape=jax.ShapeDtypeStruct((_N, 192), jnp.float32),
    )(h, degs)


def _tc_gcn_dense(h, aggp, degd, p):
    B = 600

    def body(h_r, a_r, d_r, impW, impB, decW, decB, mask,
             g2Wl, g2Wr, g0Wl, g0Wr, g0res,
             fs2_o, fd2_o, fs0_o, fd0_o, res0_o):
        agg = (a_r[0] + a_r[1]) * _deg_norm(d_r)
        h1 = _dot(agg, impW[...]) + impB[...]
        h1 = _dot(h1, decW[...]) + decB[...]
        hm = 0.1 * h_r[...] + 0.9 * h1
        l1 = jnp.sum(jnp.abs(hm), axis=1, keepdims=True)
        hm = hm / jnp.maximum(l1, 1e-12) * mask[...]
        zpad = jnp.zeros((hm.shape[0], 64), jnp.float32)
        fs2_o[...] = jnp.concatenate([_dot(hm, g2Wl[...]), zpad], axis=1)
        fd2_o[...] = jnp.concatenate([_dot(hm, g2Wr[...]), zpad], axis=1)
        fs0_o[...] = _pack_bf16(_dot(hm, g0Wl[...]))
        fd0_o[...] = _pack_bf16(_dot(hm, g0Wr[...]))
        res0_o[...] = _dot(hm, g0res[...])

    full = lambda shape: pl.BlockSpec(shape, lambda i: (0,) * len(shape))
    row = lambda d: pl.BlockSpec((B, d), lambda i: (i, 0))
    return pl.pallas_call(
        body,
        grid=(_N // B,),
        in_specs=[row(192),
                  pl.BlockSpec((2, B, 192), lambda i: (0, i, 0)),
                  pl.BlockSpec((2, B, 16), lambda i: (0, i, 0)),
                  full((192, 192)), pl.BlockSpec((192,), lambda i: (0,)),
                  full((192, 192)), pl.BlockSpec((192,), lambda i: (0,)),
                  pl.BlockSpec((192,), lambda i: (0,)),
                  full((192, 64)), full((192, 64)),
                  full((192, 512)), full((192, 512)), full((192, 512))],
        out_specs=[row(128), row(128), row(256), row(256), row(512)],
        out_shape=[jax.ShapeDtypeStruct((_N, 128), jnp.float32),
                   jax.ShapeDtypeStruct((_N, 128), jnp.float32),
                   jax.ShapeDtypeStruct((_N, 256), jnp.float32),
                   jax.ShapeDtypeStruct((_N, 256), jnp.float32),
                   jax.ShapeDtypeStruct((_N, 512), jnp.float32)],
    )(h, aggp, degd, p['impW'], p['impB'], p['decW'], p['decB'], p['mask'],
      p['g2Wl'], p['g2Wr'], p['g0Wl'], p['g0Wr'], p['g0res'])


def _tc_edge(fsg, fdg, A, R, D, DW, W, BE, packed=False, ne=_E, goff=0):
    def body(fs_r, fd_r, a_r, r_r, wex_o):
        if packed:
            fs = _unpack_bf16(fs_r[...])
            e = fs + _unpack_bf16(fd_r[...])
        else:
            fs = fs_r[:, 0:D]
            e = fs + fd_r[:, 0:D]
        e = jnp.where(e > 0, e, 0.2 * e)
        ex = jnp.exp(_dot(e, a_r[...]))
        wex_o[:, 0:D] = fs * _dot(ex, r_r[...])
        wex_o[:, D:D + 16] = ex
        wex_o[:, D + 16:W] = jnp.zeros((BE, W - D - 16), jnp.float32)

    full = lambda shape: pl.BlockSpec(shape, lambda i: (0,) * len(shape))
    rowi = lambda d: pl.BlockSpec((BE, d), lambda i: (i + goff, 0))
    row = lambda d: pl.BlockSpec((BE, d), lambda i: (i, 0))
    return pl.pallas_call(
        body,
        grid=(ne // BE,),
        in_specs=[rowi(DW), rowi(DW), full((D, 16)), full((16, D))],
        out_specs=row(W),
        out_shape=jax.ShapeDtypeStruct((ne, W), jnp.float32),
    )(fsg, fdg, A, R)


def _tc_finish(outp, R, D, W, res=None, bias=None, relu=False,
               proj=None, proj_pad=None):
    B = 600
    proj = proj or []
    proj_pad = proj_pad or [w.shape[1] for w in proj]
    nproj = len(proj)

    full = lambda shape: pl.BlockSpec(shape, lambda i: (0,) * len(shape))
    row = lambda d: pl.BlockSpec((B, d), lambda i: (i, 0))
    outps = outp if isinstance(outp, (list, tuple)) else [outp]
    in_specs = [pl.BlockSpec((2, B, W), lambda i: (0, i, 0))
                for _ in outps] + [full((16, D))]
    args = list(outps) + [R]
    if res is not None:
        in_specs.append(row(D)); args.append(res)
    if bias is not None:
        in_specs.append(pl.BlockSpec((D,), lambda i: (0,))); args.append(bias)
    for w in proj:
        in_specs.append(full(w.shape)); args.append(w)
    out_specs = [row(D)] + [row(pw) for pw in proj_pad]
    out_shape = [jax.ShapeDtypeStruct((_N, D), jnp.float32)] +                 [jax.ShapeDtypeStruct((_N, pw), jnp.float32) for pw in proj_pad]

    def body(*refs):
        nin = len(in_specs)
        ins, outs = refs[:nin], refs[nin:]
        np_ = len(outps)
        r_r = ins[np_]
        idx = np_ + 1
        dn = sum(ins[q][pp, :, D:D + 16] for q in range(np_) for pp in (0, 1))
        num = sum(ins[q][pp, :, 0:D] for q in range(np_) for pp in (0, 1))
        out = num / (_dot(dn, r_r[...]) + 1e-16)
        if res is not None:
            out = out + ins[idx][...]
            idx += 1
        if bias is not None:
            out = out + ins[idx][...]
            idx += 1
        if relu:
            out = jnp.maximum(out, 0.0)
        outs[0][...] = out
        for j in range(nproj):
            pr = _dot(out, ins[idx + j][...])
            if proj_pad[j] > pr.shape[1]:
                pr = jnp.concatenate(
                    [pr, jnp.zeros((pr.shape[0], proj_pad[j] - pr.shape[1]),
                                   jnp.float32)], axis=1)
            outs[1 + j][...] = pr

    return pl.pallas_call(
        body,
        grid=(_N // B,),
        in_specs=in_specs,
        out_specs=out_specs if len(out_specs) > 1 else out_specs[0],
        out_shape=out_shape if len(out_shape) > 1 else out_shape[0],
    )(*args)


def _tc_final(h, nf, h3, W, b):
    B = 600

    def body(h_r, n_r, h3_r, w_r, b_r, out_o):
        cat = jnp.concatenate([h_r[...], n_r[...], h3_r[...]], axis=1)
        out_o[...] = _dot(cat, w_r[...]) + b_r[...]

    full = lambda shape: pl.BlockSpec(shape, lambda i: (0,) * len(shape))
    row = lambda d: pl.BlockSpec((B, d), lambda i: (i, 0))
    return pl.pallas_call(
        body,
        grid=(_N // B,),
        in_specs=[row(64), row(16), row(64), full((144, 6)),
                  pl.BlockSpec((6,), lambda i: (0,))],
        out_specs=row(6),
        out_shape=jax.ShapeDtypeStruct((_N, 6), jnp.float32),
    )(h, nf, h3, W, b)


def _attn_mats(attn, heads, d):
    D = heads * d
    eye = jnp.eye(heads, dtype=jnp.float32)
    A = (attn[:, :, None] * eye[:, None, :]).reshape(D, heads)
    R = jnp.repeat(eye, d, axis=1)
    return A, R


def kernel(text, audio, vision, oText, oAudio, oVision, edge_index, params):
    p = params
    src = edge_index[0]
    dst = edge_index[1]
    ones16 = jnp.ones((_C, 16), jnp.float32)
    src2 = src.reshape(_NW, _NCH, _C)
    dst2 = dst.reshape(_NW, _NCH, _C)
    z16 = jnp.zeros((_RPT, 16), jnp.float32)
    z192 = jnp.zeros((_RPT, 192), jnp.float32)
    z128 = jnp.zeros((_RPT, 128), jnp.float32)

    degs, degd = _sc_degrees(src2, dst2, ones16, z16)

    stack, xf, xb = _tc_fusion(text, audio, vision, p)
    xf_t = xf.reshape(50, 120, 32).transpose(1, 0, 2)
    xb_t = xb.reshape(50, 120, 32).transpose(1, 0, 2)
    hcat = _tc_lstm(xf_t, xb_t, p['lfWhh'], p['lbWhh'])
    newF = hcat.transpose(1, 0, 2).reshape(_N, 16)

    hs = _tc_scale_h(stack, degs)
    aggp = _sc_gather_scatter(hs, src2, dst2, z192, 192)
    fs2, fd2, fs0, fd0, res0 = _tc_gcn_dense(stack, aggp, degd, p)

    A2, R2 = _attn_mats(p['g2attn'], 16, 4)
    fsg2, fdg2 = _sc_gather2(fs2, fd2, src2, dst2, 128, _C)

    A0, R0 = _attn_mats(p['g0attn'], 16, 32)
    A0 = A0 + newF[0, 0] * 0.0
    fsg, fdg = _sc_gather2(fs0, fd0, src2, dst2, 256, _C)
    wex = _tc_edge(fsg, fdg, A0, R0, 512, 256, 640, 2000, packed=True)
    outp = _sc_scatter_wex(wex, dst2, z128, 640, 5)
    h0, fs1, fd1, res1 = _tc_finish(outp, R0, 512, 640,
                                    res=res0, bias=p['g0bias'], relu=True,
                                    proj=[p['g1Wl'], p['g1Wr'], p['g1res']],
                                    proj_pad=[128, 128, 64])

    wex2 = _tc_edge(fsg2, fdg2, A2, R2, 64, 128, 128, 3000)
    outp2 = _sc_scatter_wex(wex2, dst2, z128, 128, 1)
    h3 = _tc_finish(outp2, R2, 64, 128)

    A1, R1 = _attn_mats(p['g1attn'], 16, 4)
    fsg, fdg = _sc_gather2(fs1, fd1, src2, dst2, 128, _C)
    wex = _tc_edge(fsg, fdg, A1, R1, 64, 128, 128, 3000)
    outp = _sc_scatter_wex(wex, dst2, z128, 128, 1)
    hfin = _tc_finish(outp, R1, 64, 128, res=res1, bias=p['g1bias'], relu=True)

    return _tc_final(hfin, newF, h3, p['linW'], p['linB'])


# speedup vs baseline: 1.0621x; 1.0048x over previous
import functools

import jax
import jax.numpy as jnp
from jax import lax
from jax.experimental import pallas as pl
from jax.experimental.pallas import tpu as pltpu
from jax.experimental.pallas import tpu_sc as plsc

_N = 6000
_E = 96000
_NC = 2
_NS = 16
_NW = _NC * _NS
_EPT = _E // _NW
_C = 120
_NCH = _EPT // _C
_NP = 6016
_RPT = _NP // _NS

_mesh = plsc.VectorSubcoreMesh(core_axis_name="c", subcore_axis_name="s")


def _wid():
    return lax.axis_index("s") * _NC + lax.axis_index("c")


def _sc_degrees(srcr, dstr, ones16, zeros16):
    @functools.partial(
        pl.kernel, mesh=_mesh,
        compiler_params=pltpu.CompilerParams(use_tc_tiling_on_sc=False),
        out_type=(jax.ShapeDtypeStruct((_NC, _NP, 16), jnp.float32),
                  jax.ShapeDtypeStruct((_NC, _NP, 16), jnp.float32)),
        scratch_types=[pltpu.VMEM((_NCH, _C), jnp.int32),
                       pltpu.VMEM((_NCH, _C), jnp.int32),
                       pltpu.VMEM((_C, 16), jnp.float32),
                       pltpu.VMEM_SHARED((_NP, 16), jnp.float32),
                       pltpu.VMEM_SHARED((_NP, 16), jnp.float32),
                       pltpu.SemaphoreType.DMA, pltpu.SemaphoreType.DMA],
    )
    def k(src_h, dst_h, ones_h, zeros_h, degs_o, degd_o, ivs, ivd, vones,
          acca, accb, sa, sb):
        c = lax.axis_index("c")
        s = lax.axis_index("s")
        wid = _wid()
        rbase = s * _RPT
        pltpu.sync_copy(src_h.at[wid], ivs)
        pltpu.sync_copy(dst_h.at[wid], ivd)
        pltpu.sync_copy(ones_h, vones)
        pltpu.sync_copy(zeros_h, acca.at[pl.ds(rbase, _RPT)])
        pltpu.sync_copy(zeros_h, accb.at[pl.ds(rbase, _RPT)])
        plsc.subcore_barrier()

        def fire(j, carry):
            pltpu.async_copy(vones, acca.at[ivs.at[j]], sa, add=True)
            pltpu.async_copy(vones, accb.at[ivd.at[j]], sb, add=True)
            return carry
        lax.fori_loop(0, _NCH, fire, 0)

        def drain(j, carry):
            pltpu.make_async_copy(vones, acca.at[pl.ds(0, _C)], sa).wait()
            pltpu.make_async_copy(vones, accb.at[pl.ds(0, _C)], sb).wait()
            return carry
        lax.fori_loop(0, _NCH, drain, 0)
        plsc.subcore_barrier()
        pltpu.sync_copy(acca.at[pl.ds(rbase, _RPT)],
                        degs_o.at[c, pl.ds(rbase, _RPT)])
        pltpu.sync_copy(accb.at[pl.ds(rbase, _RPT)],
                        degd_o.at[c, pl.ds(rbase, _RPT)])
    return k(srcr, dstr, ones16, zeros16)


def _sc_gather_scatter(table, gidxr, sidxr, zeros, D):
    @functools.partial(
        pl.kernel, mesh=_mesh,
        compiler_params=pltpu.CompilerParams(use_tc_tiling_on_sc=False),
        out_type=jax.ShapeDtypeStruct((_NC, _NP, D), jnp.float32),
        scratch_types=[pltpu.VMEM((_NCH, _C), jnp.int32),
                       pltpu.VMEM((_NCH, _C), jnp.int32),
                       pltpu.VMEM((_C, D), jnp.float32),
                       pltpu.VMEM((_C, D), jnp.float32),
                       pltpu.VMEM_SHARED((_NP, D), jnp.float32),
                       pltpu.SemaphoreType.DMA, pltpu.SemaphoreType.DMA,
                       pltpu.SemaphoreType.DMA, pltpu.SemaphoreType.DMA],
    )
    def k(tab_h, gi_h, si_h, zeros_h, out_o, ivg, ivd, ra, rb, acc,
          sga, sgb, ssa, ssb):
        c = lax.axis_index("c")
        s = lax.axis_index("s")
        wid = _wid()
        rbase = s * _RPT
        pltpu.sync_copy(gi_h.at[wid], ivg)
        pltpu.sync_copy(si_h.at[wid], ivd)
        pltpu.sync_copy(zeros_h, acc.at[pl.ds(rbase, _RPT)])
        plsc.subcore_barrier()
        r = (ra, rb)
        sg = (sga, sgb)
        ss = (ssa, ssb)

        def fire_gather(j, k_):
            pltpu.async_copy(tab_h.at[ivg.at[j]], r[k_], sg[k_])

        def drain_scatter(k_):
            pltpu.make_async_copy(r[k_], acc.at[pl.ds(0, _C)], ss[k_]).wait()

        def finish(j, k_):
            pltpu.make_async_copy(tab_h.at[ivg.at[j]], r[k_], sg[k_]).wait()
            pltpu.async_copy(r[k_], acc.at[ivd.at[j]], ss[k_], add=True)

        fire_gather(0, 0)
        fire_gather(1, 1)
        finish(0, 0)

        def body(j0, carry):
            j1 = 2 * j0 + 1
            drain_scatter(0)
            fire_gather(j1 + 1, 0)
            finish(j1, 1)
            drain_scatter(1)
            fire_gather(j1 + 2, 1)
            finish(j1 + 1, 0)
            return carry
        lax.fori_loop(0, (_NCH - 3) // 2, body, 0)
        drain_scatter(0)
        fire_gather(_NCH - 1, 0)
        finish(_NCH - 2, 1)
        finish(_NCH - 1, 0)
        drain_scatter(0)
        drain_scatter(1)
        plsc.subcore_barrier()
        pltpu.sync_copy(acc.at[pl.ds(rbase, _RPT)],
                        out_o.at[c, pl.ds(rbase, _RPT)])
    return k(table, gidxr, sidxr, zeros)


def _sc_gather2(t1, t2, i1r, i2r, D, C):
    @functools.partial(
        pl.kernel, mesh=_mesh,
        out_type=(jax.ShapeDtypeStruct((_E, D), jnp.float32),
                  jax.ShapeDtypeStruct((_E, D), jnp.float32)),
        scratch_types=[pltpu.VMEM((_EPT // C, C), jnp.int32),
                       pltpu.VMEM((_EPT // C, C), jnp.int32),
                       pltpu.VMEM((C, D), jnp.float32),
                       pltpu.VMEM((C, D), jnp.float32),
                       pltpu.VMEM((C, D), jnp.float32),
                       pltpu.VMEM((C, D), jnp.float32),
                       pltpu.SemaphoreType.DMA, pltpu.SemaphoreType.DMA,
                       pltpu.SemaphoreType.DMA, pltpu.SemaphoreType.DMA,
                       pltpu.SemaphoreType.DMA, pltpu.SemaphoreType.DMA,
                       pltpu.SemaphoreType.DMA, pltpu.SemaphoreType.DMA],
    )
    def k(t1_h, t2_h, i1_h, i2_h, o1_o, o2_o, iv1, iv2, r1a, r1b, r2a, r2b,
          sg1a, sg1b, sg2a, sg2b, sw1a, sw1b, sw2a, sw2b):
        wid = _wid()
        base = wid * _EPT
        pltpu.sync_copy(i1_h.at[wid], iv1)
        pltpu.sync_copy(i2_h.at[wid], iv2)
        r1 = (r1a, r1b)
        r2 = (r2a, r2b)
        sg1 = (sg1a, sg1b)
        sg2 = (sg2a, sg2b)
        sw1 = (sw1a, sw1b)
        sw2 = (sw2a, sw2b)

        def chunk(j, k, drain):
            if drain:
                pltpu.make_async_copy(r1[k], o1_o.at[pl.ds(0, C)], sw1[k]).wait()
                pltpu.make_async_copy(r2[k], o2_o.at[pl.ds(0, C)], sw2[k]).wait()
            g1 = pltpu.async_copy(t1_h.at[iv1.at[j]], r1[k], sg1[k])
            g2 = pltpu.async_copy(t2_h.at[iv2.at[j]], r2[k], sg2[k])
            off = base + j * C
            g1.wait()
            pltpu.async_copy(r1[k], o1_o.at[pl.ds(off, C)], sw1[k])
            g2.wait()
            pltpu.async_copy(r2[k], o2_o.at[pl.ds(off, C)], sw2[k])

        chunk(0, 0, False)
        chunk(1, 1, False)

        def body(j0, carry):
            j = 2 * j0
            chunk(j, 0, True)
            chunk(j + 1, 1, True)
            return carry
        lax.fori_loop(1, (_EPT // C - 1) // 2, body, 0)
        chunk(_EPT // C - 1, 0, True)
        pltpu.make_async_copy(r1[0], o1_o.at[pl.ds(0, C)], sw1[0]).wait()
        pltpu.make_async_copy(r2[0], o2_o.at[pl.ds(0, C)], sw2[0]).wait()
        pltpu.make_async_copy(r1[1], o1_o.at[pl.ds(0, C)], sw1[1]).wait()
        pltpu.make_async_copy(r2[1], o2_o.at[pl.ds(0, C)], sw2[1]).wait()
    return k(t1, t2, i1r, i2r)


def _sc_scatter_wex(wex, sidxr, zeros, W, SPLIT, NCH=_NCH):
    DS = W // SPLIT
    EPT = NCH * _C
    M = _NW * EPT

    @functools.partial(
        pl.kernel, mesh=_mesh,
        out_type=jax.ShapeDtypeStruct((_NC, _NP, W), jnp.float32),
        scratch_types=[pltpu.VMEM((NCH, _C), jnp.int32),
                       pltpu.VMEM((_C, DS), jnp.float32),
                       pltpu.VMEM((_C, DS), jnp.float32),
                       pltpu.VMEM_SHARED((_NP, DS), jnp.float32),
                       pltpu.SemaphoreType.DMA, pltpu.SemaphoreType.DMA,
                       pltpu.SemaphoreType.DMA, pltpu.SemaphoreType.DMA],
    )
    def k(w_h, si_h, zeros_h, out_o, iv, vwa, vwb, accd, sva, svb, ssa, ssb):
        c = lax.axis_index("c")
        s_ = lax.axis_index("s")
        wid = s_ * _NC + c
        base = wid * EPT
        rbase = s_ * _RPT
        pltpu.sync_copy(si_h.at[wid], iv)
        vw = (vwa, vwb)
        sv = (sva, svb)
        ss = (ssa, ssb)
        for sp in range(SPLIT):
            pltpu.sync_copy(zeros_h, accd.at[pl.ds(rbase, _RPT)])
            plsc.subcore_barrier()

            def fire_load(j, k):
                off = base + j * _C
                pltpu.async_copy(
                    w_h.at[pl.ds(off, _C), pl.ds(sp * DS, DS)], vw[k], sv[k])

            def drain_scatter(k):
                pltpu.make_async_copy(
                    vw[k], accd.at[pl.ds(0, _C)], ss[k]).wait()

            def finish(j, k):
                pltpu.make_async_copy(
                    w_h.at[pl.ds(0, _C), pl.ds(sp * DS, DS)], vw[k],
                    sv[k]).wait()
                pltpu.async_copy(vw[k], accd.at[iv.at[j]], ss[k], add=True)

            fire_load(0, 0)
            fire_load(1, 1)
            finish(0, 0)

            def body(j0, carry):
                j1 = 2 * j0 + 1
                drain_scatter(0)
                fire_load(j1 + 1, 0)
                finish(j1, 1)
                drain_scatter(1)
                fire_load(j1 + 2, 1)
                finish(j1 + 1, 0)
                return carry
            lax.fori_loop(0, (NCH - 3) // 2, body, 0)
            if NCH % 2 == 1:
                drain_scatter(0)
                fire_load(NCH - 1, 0)
                finish(NCH - 2, 1)
                finish(NCH - 1, 0)
            else:
                drain_scatter(0)
                fire_load(NCH - 2, 0)
                finish(NCH - 3, 1)
                drain_scatter(1)
                fire_load(NCH - 1, 1)
                finish(NCH - 2, 0)
                finish(NCH - 1, 1)
            drain_scatter(0)
            drain_scatter(1)
            plsc.subcore_barrier()
            pltpu.sync_copy(accd.at[pl.ds(rbase, _RPT)],
                            out_o.at[c, pl.ds(rbase, _RPT), pl.ds(sp * DS, DS)])
            if sp < SPLIT - 1:
                plsc.subcore_barrier()
    return k(wex, sidxr, zeros)


def _dot(a, b):
    return jnp.dot(a, b, preferred_element_type=jnp.float32)


def _pack_bf16(v):
    h = v.shape[1] // 2
    vi = lax.bitcast_convert_type(v, jnp.int32)
    vr = vi + (0x7FFF + (lax.shift_right_logical(vi, 16) & 1))
    hi = vr[:, 0:h] & jnp.int32(-65536)
    lo = lax.shift_right_logical(vr[:, h:2 * h], 16)
    return lax.bitcast_convert_type(hi | lo, jnp.float32)


def _unpack_bf16(pv):
    pi = lax.bitcast_convert_type(pv, jnp.int32)
    a = lax.bitcast_convert_type(pi & jnp.int32(-65536), jnp.float32)
    b = lax.bitcast_convert_type(lax.shift_left(pi, 16), jnp.float32)
    return jnp.concatenate([a, b], axis=1)


def _tc_fusion(text, audio, vision, p):
    B = 600

    def body(t_r, a_r, v_r, tw, tb, aw, ab, vw, vb, wf, bf1, bf2, wb, bb1, bb2,
             stack_o, xf_o, xb_o):
        t = _dot(t_r[...], tw[...]) + tb[...]
        a = _dot(a_r[...], aw[...]) + ab[...]
        v = _dot(v_r[...], vw[...]) + vb[...]
        stack = jnp.concatenate([t, a, v], axis=1)
        stack_o[...] = stack
        xf_o[...] = _dot(stack, wf[...]) + bf1[...] + bf2[...]
        xb_o[...] = _dot(stack, wb[...]) + bb1[...] + bb2[...]

    full = lambda shape: pl.BlockSpec(shape, lambda i: (0,) * len(shape))
    row = lambda d: pl.BlockSpec((B, d), lambda i: (i, 0))
    vec = lambda d: pl.BlockSpec((d,), lambda i: (0,))
    return pl.pallas_call(
        body,
        grid=(_N // B,),
        in_specs=[row(1024), row(512), row(1024),
                  full((1024, 64)), vec(64), full((512, 64)), vec(64),
                  full((1024, 64)), vec(64),
                  full((192, 32)), vec(32), vec(32),
                  full((192, 32)), vec(32), vec(32)],
        out_specs=[row(192), row(32), row(32)],
        out_shape=[jax.ShapeDtypeStruct((_N, 192), jnp.float32),
                   jax.ShapeDtypeStruct((_N, 32), jnp.float32),
                   jax.ShapeDtypeStruct((_N, 32), jnp.float32)],
    )(text, audio, vision,
      p['textW'], p['textB'], p['audioW'], p['audioB'], p['visionW'], p['visionB'],
      p['lfWih'], p['lfbih'], p['lfbhh'], p['lbWih'], p['lbbih'], p['lbbhh'])


def _tc_lstm(xf, xb, whf, whb):
    T, Bb, H = 120, 50, 8

    def body(xf_r, xb_r, wf_r, wb_r, out_o):
        def gates(g, c):
            i, f, gg, o = jnp.split(g, 4, axis=-1)
            c2 = jax.nn.sigmoid(f) * c + jax.nn.sigmoid(i) * jnp.tanh(gg)
            h2 = jax.nn.sigmoid(o) * jnp.tanh(c2)
            return h2, c2

        def fstep(t, hc):
            h, c = hc
            h2, c2 = gates(xf_r[t] + _dot(h, wf_r[...]), c)
            out_o[t, :, 0:8] = h2
            return (h2, c2)

        def bstep(t, hc):
            h, c = hc
            tt = T - 1 - t
            h2, c2 = gates(xb_r[tt] + _dot(h, wb_r[...]), c)
            out_o[tt, :, 8:16] = h2
            return (h2, c2)

        z = (jnp.zeros((Bb, H), jnp.float32), jnp.zeros((Bb, H), jnp.float32))
        lax.fori_loop(0, T, fstep, z)
        lax.fori_loop(0, T, bstep, z)

    return pl.pallas_call(
        body,
        out_shape=jax.ShapeDtypeStruct((T, Bb, 16), jnp.float32),
    )(xf, xb, whf, whb)


def _deg_norm(dp):
    deg = dp[0, :, 0:1] + dp[1, :, 0:1]
    return jnp.where(deg > 0, lax.rsqrt(jnp.maximum(deg, 1.0)), 0.0)


def _tc_scale_h(h, degs):
    B = 600

    def body(h_r, d_r, out_o):
        out_o[...] = h_r[...] * _deg_norm(d_r)

    return pl.pallas_call(
        body,
        grid=(_N // B,),
        in_specs=[pl.BlockSpec((B, 192), lambda i: (i, 0)),
                  pl.BlockSpec((2, B, 16), lambda i: (0, i, 0))],
        out_specs=pl.BlockSpec((B, 192), lambda i: (i, 0)),
        out_shape=jax.ShapeDtypeStruct((_N, 192), jnp.float32),
    )(h, degs)


def _tc_gcn_dense(h, aggp, degd, p):
    B = 600

    def body(h_r, a_r, d_r, impW, impB, decW, decB, mask,
             g2Wl, g2Wr, g0Wl, g0Wr, g0res,
             fs2_o, fd2_o, fs0_o, fd0_o, res0_o):
        agg = (a_r[0] + a_r[1]) * _deg_norm(d_r)
        h1 = _dot(agg, impW[...]) + impB[...]
        h1 = _dot(h1, decW[...]) + decB[...]
        hm = 0.1 * h_r[...] + 0.9 * h1
        l1 = jnp.sum(jnp.abs(hm), axis=1, keepdims=True)
        hm = hm / jnp.maximum(l1, 1e-12) * mask[...]
        zpad = jnp.zeros((hm.shape[0], 64), jnp.float32)
        fs2_o[...] = jnp.concatenate([_dot(hm, g2Wl[...]), zpad], axis=1)
        fd2_o[...] = jnp.concatenate([_dot(hm, g2Wr[...]), zpad], axis=1)
        fs0_o[...] = _pack_bf16(_dot(hm, g0Wl[...]))
        fd0_o[...] = _pack_bf16(_dot(hm, g0Wr[...]))
        res0_o[...] = _dot(hm, g0res[...])

    full = lambda shape: pl.BlockSpec(shape, lambda i: (0,) * len(shape))
    row = lambda d: pl.BlockSpec((B, d), lambda i: (i, 0))
    return pl.pallas_call(
        body,
        grid=(_N // B,),
        in_specs=[row(192),
                  pl.BlockSpec((2, B, 192), lambda i: (0, i, 0)),
                  pl.BlockSpec((2, B, 16), lambda i: (0, i, 0)),
                  full((192, 192)), pl.BlockSpec((192,), lambda i: (0,)),
                  full((192, 192)), pl.BlockSpec((192,), lambda i: (0,)),
                  pl.BlockSpec((192,), lambda i: (0,)),
                  full((192, 64)), full((192, 64)),
                  full((192, 512)), full((192, 512)), full((192, 512))],
        out_specs=[row(128), row(128), row(256), row(256), row(512)],
        out_shape=[jax.ShapeDtypeStruct((_N, 128), jnp.float32),
                   jax.ShapeDtypeStruct((_N, 128), jnp.float32),
                   jax.ShapeDtypeStruct((_N, 256), jnp.float32),
                   jax.ShapeDtypeStruct((_N, 256), jnp.float32),
                   jax.ShapeDtypeStruct((_N, 512), jnp.float32)],
    )(h, aggp, degd, p['impW'], p['impB'], p['decW'], p['decB'], p['mask'],
      p['g2Wl'], p['g2Wr'], p['g0Wl'], p['g0Wr'], p['g0res'])


def _tc_edge(fsg, fdg, A, R, D, DW, W, BE, packed=False, ne=_E, goff=0):
    def body(fs_r, fd_r, a_r, r_r, wex_o):
        if packed:
            fs = _unpack_bf16(fs_r[...])
            e = fs + _unpack_bf16(fd_r[...])
        else:
            fs = fs_r[:, 0:D]
            e = fs + fd_r[:, 0:D]
        e = jnp.where(e > 0, e, 0.2 * e)
        ex = jnp.exp(_dot(e, a_r[...]))
        wex_o[:, 0:D] = fs * _dot(ex, r_r[...])
        wex_o[:, D:D + 16] = ex
        wex_o[:, D + 16:W] = jnp.zeros((BE, W - D - 16), jnp.float32)

    full = lambda shape: pl.BlockSpec(shape, lambda i: (0,) * len(shape))
    rowi = lambda d: pl.BlockSpec((BE, d), lambda i: (i + goff, 0))
    row = lambda d: pl.BlockSpec((BE, d), lambda i: (i, 0))
    return pl.pallas_call(
        body,
        grid=(ne // BE,),
        in_specs=[rowi(DW), rowi(DW), full((D, 16)), full((16, D))],
        out_specs=row(W),
        out_shape=jax.ShapeDtypeStruct((ne, W), jnp.float32),
    )(fsg, fdg, A, R)


def _tc_finish(outp, R, D, W, res=None, bias=None, relu=False,
               proj=None, proj_pad=None):
    B = 600
    proj = proj or []
    proj_pad = proj_pad or [w.shape[1] for w in proj]
    nproj = len(proj)

    full = lambda shape: pl.BlockSpec(shape, lambda i: (0,) * len(shape))
    row = lambda d: pl.BlockSpec((B, d), lambda i: (i, 0))
    outps = outp if isinstance(outp, (list, tuple)) else [outp]
    in_specs = [pl.BlockSpec((2, B, W), lambda i: (0, i, 0))
                for _ in outps] + [full((16, D))]
    args = list(outps) + [R]
    if res is not None:
        in_specs.append(row(D)); args.append(res)
    if bias is not None:
        in_specs.append(pl.BlockSpec((D,), lambda i: (0,))); args.append(bias)
    for w in proj:
        in_specs.append(full(w.shape)); args.append(w)
    out_specs = [row(D)] + [row(pw) for pw in proj_pad]
    out_shape = [jax.ShapeDtypeStruct((_N, D), jnp.float32)] +                 [jax.ShapeDtypeStruct((_N, pw), jnp.float32) for pw in proj_pad]

    def body(*refs):
        nin = len(in_specs)
        ins, outs = refs[:nin], refs[nin:]
        np_ = len(outps)
        r_r = ins[np_]
        idx = np_ + 1
        dn = sum(ins[q][pp, :, D:D + 16] for q in range(np_) for pp in (0, 1))
        num = sum(ins[q][pp, :, 0:D] for q in range(np_) for pp in (0, 1))
        out = num / (_dot(dn, r_r[...]) + 1e-16)
        if res is not None:
            out = out + ins[idx][...]
            idx += 1
        if bias is not None:
            out = out + ins[idx][...]
            idx += 1
        if relu:
            out = jnp.maximum(out, 0.0)
        outs[0][...] = out
        for j in range(nproj):
            pr = _dot(out, ins[idx + j][...])
            if proj_pad[j] > pr.shape[1]:
                pr = jnp.concatenate(
                    [pr, jnp.zeros((pr.shape[0], proj_pad[j] - pr.shape[1]),
                                   jnp.float32)], axis=1)
            outs[1 + j][...] = pr

    return pl.pallas_call(
        body,
        grid=(_N // B,),
        in_specs=in_specs,
        out_specs=out_specs if len(out_specs) > 1 else out_specs[0],
        out_shape=out_shape if len(out_shape) > 1 else out_shape[0],
    )(*args)


def _tc_final(h, nf, h3, W, b):
    B = 600

    def body(h_r, n_r, h3_r, w_r, b_r, out_o):
        cat = jnp.concatenate([h_r[...], n_r[...], h3_r[...]], axis=1)
        out_o[...] = _dot(cat, w_r[...]) + b_r[...]

    full = lambda shape: pl.BlockSpec(shape, lambda i: (0,) * len(shape))
    row = lambda d: pl.BlockSpec((B, d), lambda i: (i, 0))
    return pl.pallas_call(
        body,
        grid=(_N // B,),
        in_specs=[row(64), row(16), row(64), full((144, 6)),
                  pl.BlockSpec((6,), lambda i: (0,))],
        out_specs=row(6),
        out_shape=jax.ShapeDtypeStruct((_N, 6), jnp.float32),
    )(h, nf, h3, W, b)


def _attn_mats(attn, heads, d):
    D = heads * d
    eye = jnp.eye(heads, dtype=jnp.float32)
    A = (attn[:, :, None] * eye[:, None, :]).reshape(D, heads)
    R = jnp.repeat(eye, d, axis=1)
    return A, R


def kernel(text, audio, vision, oText, oAudio, oVision, edge_index, params):
    p = params
    src = edge_index[0]
    dst = edge_index[1]
    ones16 = jnp.ones((_C, 16), jnp.float32)
    src2 = src.reshape(_NW, _NCH, _C)
    dst2 = dst.reshape(_NW, _NCH, _C)
    z16 = jnp.zeros((_RPT, 16), jnp.float32)
    z192 = jnp.zeros((_RPT, 192), jnp.float32)
    z128 = jnp.zeros((_RPT, 128), jnp.float32)

    degs, degd = _sc_degrees(src2, dst2, ones16, z16)

    stack, xf, xb = _tc_fusion(text, audio, vision, p)
    xf_t = xf.reshape(50, 120, 32).transpose(1, 0, 2)
    xb_t = xb.reshape(50, 120, 32).transpose(1, 0, 2)
    hcat = _tc_lstm(xf_t, xb_t, p['lfWhh'], p['lbWhh'])
    newF = hcat.transpose(1, 0, 2).reshape(_N, 16)

    hs = _tc_scale_h(stack, degs)
    aggp = _sc_gather_scatter(hs, src2, dst2, z192, 192)
    fs2, fd2, fs0, fd0, res0 = _tc_gcn_dense(stack, aggp, degd, p)

    A2, R2 = _attn_mats(p['g2attn'], 16, 4)
    fsg2, fdg2 = _sc_gather2(fs2, fd2, src2, dst2, 128, _C)

    A0, R0 = _attn_mats(p['g0attn'], 16, 32)
    A0 = A0 + newF[0, 0] * 0.0
    fsg, fdg = _sc_gather2(fs0, fd0, src2, dst2, 256, _C)
    wex = _tc_edge(fsg, fdg, A0, R0, 512, 256, 640, 2000, packed=True)
    outp = _sc_scatter_wex(wex, dst2, z128, 640, 5)
    h0, fs1, fd1, res1 = _tc_finish(outp, R0, 512, 640,
                                    res=res0, bias=p['g0bias'], relu=True,
                                    proj=[p['g1Wl'], p['g1Wr'], p['g1res']],
                                    proj_pad=[128, 128, 64])

    wex2 = _tc_edge(fsg2, fdg2, A2, R2, 64, 128, 128, 4800)
    outp2 = _sc_scatter_wex(wex2, dst2, z128, 128, 1)
    h3 = _tc_finish(outp2, R2, 64, 128)

    A1, R1 = _attn_mats(p['g1attn'], 16, 4)
    fsg, fdg = _sc_gather2(fs1, fd1, src2, dst2, 128, _C)
    wex = _tc_edge(fsg, fdg, A1, R1, 64, 128, 128, 4800)
    outp = _sc_scatter_wex(wex, dst2, z128, 128, 1)
    hfin = _tc_finish(outp, R1, 64, 128, res=res1, bias=p['g1bias'], relu=True)

    return _tc_final(hfin, newF, h3, p['linW'], p['linB'])
